# Initial kernel scaffold; baseline (speedup 1.0000x reference)
import jax
import jax.numpy as jnp
from jax.experimental import pallas as pl


def kernel(h, x, edge_index, edge_dist, We1, be1, We2, be2, Wn1, bn1, Wn2, bn2, Wc1, bc1, Wc2):
    raise NotImplementedError("write your pallas kernel here")


# speedup vs baseline: 3.2507x; 3.2507x over previous
import functools

import jax
import jax.numpy as jnp
from jax import lax
from jax.experimental import pallas as pl
from jax.experimental.pallas import tpu as pltpu
from jax.experimental.pallas import tpu_sc as plsc

_NC = 2
_NS = 16
_NW = _NC * _NS

_WT = 256
_WU = 128
_CH = 80


def _sc_mesh():
    return plsc.VectorSubcoreMesh(
        core_axis_name="c", subcore_axis_name="s",
        num_cores=_NC, num_subcores=_NS)


def _gather_sc(t, src, dst, n, e):
    epw = e // _NW
    cpt = epw // _CH

    @functools.partial(
        pl.kernel,
        out_type=(
            jax.ShapeDtypeStruct((e, _WT), jnp.float32),
            jax.ShapeDtypeStruct((e, _WT), jnp.float32),
        ),
        mesh=_sc_mesh(),
        scratch_types=[
            pltpu.VMEM((_CH,), jnp.int32),
            pltpu.VMEM((_CH, _WT), jnp.float32),
            pltpu.SemaphoreType.DMA,
        ],
    )
    def k(t_hbm, src_hbm, dst_hbm, ts_hbm, td_hbm, idx_v, rows_v, sem):
        wid = lax.axis_index("s") * _NC + lax.axis_index("c")
        base = wid * epw

        def chunk(i, _):
            off = base + i * _CH
            pltpu.sync_copy(src_hbm.at[pl.ds(off, _CH)], idx_v)
            pltpu.async_copy(t_hbm.at[idx_v], rows_v, sem).wait()
            pltpu.sync_copy(rows_v, ts_hbm.at[pl.ds(off, _CH)])
            pltpu.sync_copy(dst_hbm.at[pl.ds(off, _CH)], idx_v)
            pltpu.async_copy(t_hbm.at[idx_v], rows_v, sem).wait()
            pltpu.sync_copy(rows_v, td_hbm.at[pl.ds(off, _CH)])
            return _

        lax.fori_loop(0, cpt, chunk, 0)

    return k(t, src, dst)


def _scatter_sc(u1, u2, dst, zeros_nw, n, e):
    epw = e // _NW
    cpt = epw // _CH
    rpt = (n // _NS) // 8 * 8
    rem = n - _NS * rpt

    @functools.partial(
        pl.kernel,
        out_type=(
            jax.ShapeDtypeStruct((_NC, n, _WU), jnp.float32),
            jax.ShapeDtypeStruct((_NC, n, _WU), jnp.float32),
        ),
        mesh=_sc_mesh(),
        scratch_types=[
            pltpu.VMEM((_CH,), jnp.int32),
            pltpu.VMEM((_CH, _WU), jnp.float32),
            pltpu.VMEM_SHARED((n, _WU), jnp.float32),
            pltpu.SemaphoreType.DMA,
        ],
    )
    def k(u1_hbm, u2_hbm, dst_hbm, z_hbm, s1_hbm, s2_hbm,
          idx_v, rows_v, acc, sem):
        c = lax.axis_index("c")
        s = lax.axis_index("s")
        wid = s * _NC + c
        base = wid * epw
        mine = pl.ds(s * rpt, rpt)
        tail = pl.ds(_NS * rpt, rem)

        def phase(u_hbm, out_hbm):
            pltpu.sync_copy(z_hbm.at[mine], acc.at[mine])
            @pl.when(s == _NS - 1)
            def _zero_tail():
                pltpu.sync_copy(z_hbm.at[tail], acc.at[tail])
            plsc.subcore_barrier()

            def chunk(i, _):
                off = base + i * _CH
                pltpu.sync_copy(dst_hbm.at[pl.ds(off, _CH)], idx_v)
                pltpu.sync_copy(u_hbm.at[pl.ds(off, _CH)], rows_v)
                pltpu.sync_copy(rows_v, acc.at[idx_v], add=True)
                return _

            lax.fori_loop(0, cpt, chunk, 0)
            plsc.subcore_barrier()
            pltpu.sync_copy(acc.at[mine], out_hbm.at[c].at[mine])
            @pl.when(s == _NS - 1)
            def _write_tail():
                pltpu.sync_copy(acc.at[tail], out_hbm.at[c].at[tail])
            plsc.subcore_barrier()

        phase(u1_hbm, s1_hbm)
        phase(u2_hbm, s2_hbm)

    return k(u1, u2, dst, zeros_nw)


def _silu(v):
    return v * jax.nn.sigmoid(v)


def _edge_tc(ts, td, dist3, We1r, be1r, We2, be2r, Wn1s, Wn1d, Wn1e, bn1r,
             Wc1s, Wc1d, Wc1e, bc1r, Wc2r, e, be):
    g = e // be

    def body(ts_ref, td_ref, d_ref, we1_ref, be1_ref, we2_ref, be2_ref,
             wn1s_ref, wn1d_ref, wn1e_ref, bn1_ref,
             wc1s_ref, wc1d_ref, wc1e_ref, bc1_ref, wc2_ref,
             u1_ref, u2_ref):
        d = d_ref[0, 0, :]
        e1 = d[:, None] * we1_ref[0, :][None, :] + be1_ref[0, :]
        ea = jnp.dot(_silu(e1), we2_ref[...],
                     preferred_element_type=jnp.float32) + be2_ref[0, :]
        hs = ts_ref[:, 0:128]
        hd = td_ref[:, 0:128]
        pre_n = (jnp.dot(hs, wn1s_ref[...], preferred_element_type=jnp.float32)
                 + jnp.dot(hd, wn1d_ref[...], preferred_element_type=jnp.float32)
                 + jnp.dot(ea, wn1e_ref[...], preferred_element_type=jnp.float32)
                 + bn1_ref[0, :])
        u1_ref[...] = _silu(pre_n)
        pre_c = (jnp.dot(hs, wc1s_ref[...], preferred_element_type=jnp.float32)
                 + jnp.dot(hd, wc1d_ref[...], preferred_element_type=jnp.float32)
                 + jnp.dot(ea, wc1e_ref[...], preferred_element_type=jnp.float32)
                 + bc1_ref[0, :])
        u = _silu(pre_c)
        cw = jnp.sum(u * wc2_ref[0, :][None, :], axis=1, keepdims=True)
        dvec = ts_ref[:, 128:131] - td_ref[:, 128:131]
        dlen = jnp.maximum(
            jnp.sqrt(jnp.sum(dvec * dvec, axis=1, keepdims=True)), 1e-8)
        cu = cw * (dvec / dlen)
        ones = jnp.ones((be, 1), jnp.float32)
        pad = jnp.zeros((be, _WU - 4), jnp.float32)
        u2_ref[...] = jnp.concatenate([cu, ones, pad], axis=1)

    full = lambda shape: pl.BlockSpec(shape, lambda i: (0,) * len(shape))
    return pl.pallas_call(
        body,
        grid=(g,),
        in_specs=[
            pl.BlockSpec((be, _WT), lambda i: (i, 0)),
            pl.BlockSpec((be, _WT), lambda i: (i, 0)),
            pl.BlockSpec((1, 1, be), lambda i: (i, 0, 0)),
            full((1, 16)), full((1, 16)), full((16, 16)), full((1, 16)),
            full((128, 128)), full((128, 128)), full((16, 128)), full((1, 128)),
            full((128, 128)), full((128, 128)), full((16, 128)), full((1, 128)),
            full((1, 128)),
        ],
        out_specs=[
            pl.BlockSpec((be, _WU), lambda i: (i, 0)),
            pl.BlockSpec((be, _WU), lambda i: (i, 0)),
        ],
        out_shape=[
            jax.ShapeDtypeStruct((e, _WU), jnp.float32),
            jax.ShapeDtypeStruct((e, _WU), jnp.float32),
        ],
    )(ts, td, dist3, We1r, be1r, We2, be2r, Wn1s, Wn1d, Wn1e, bn1r,
      Wc1s, Wc1d, Wc1e, bc1r, Wc2r)


def _final_tc(h, x3, s1, s2, Wn2, bn2r, n, bn):
    g = n // bn

    def body(h_ref, x_ref, s1_ref, s2_ref, wn2_ref, bn2_ref, oh_ref, ox_ref):
        hidden = s1_ref[0] + s1_ref[1]
        s2 = s2_ref[0] + s2_ref[1]
        deg = s2[:, 3:4]
        xa = s2[:, 0:3]
        oh_ref[...] = (h_ref[...]
                       + jnp.dot(hidden, wn2_ref[...],
                                 preferred_element_type=jnp.float32)
                       + deg * bn2_ref[0, :])
        ox_ref[...] = x_ref[...] + xa

    return pl.pallas_call(
        body,
        grid=(g,),
        in_specs=[
            pl.BlockSpec((bn, 128), lambda i: (i, 0)),
            pl.BlockSpec((bn, 3), lambda i: (i, 0)),
            pl.BlockSpec((_NC, bn, _WU), lambda i: (0, i, 0)),
            pl.BlockSpec((_NC, bn, _WU), lambda i: (0, i, 0)),
            pl.BlockSpec((128, 128), lambda i: (0, 0)),
            pl.BlockSpec((1, 128), lambda i: (0, 0)),
        ],
        out_specs=[
            pl.BlockSpec((bn, 128), lambda i: (i, 0)),
            pl.BlockSpec((bn, 3), lambda i: (i, 0)),
        ],
        out_shape=[
            jax.ShapeDtypeStruct((n, 128), jnp.float32),
            jax.ShapeDtypeStruct((n, 3), jnp.float32),
        ],
    )(h, x3, s1, s2, Wn2, bn2r)


@jax.jit
def kernel(h, x, edge_index, edge_dist, We1, be1, We2, be2, Wn1, bn1, Wn2,
           bn2, Wc1, bc1, Wc2):
    n, nd = h.shape
    e = edge_dist.shape[0]
    assert nd == 128 and e % (_NW * _CH) == 0 and n % _NS == 0

    src = edge_index[0]
    dst = edge_index[1]

    t = jnp.concatenate(
        [h, x, jnp.zeros((n, _WT - nd - 3), jnp.float32)], axis=1)

    ts, td = _gather_sc(t, src, dst, n, e)

    be = 2560
    dist3 = edge_dist.reshape(e // be, 1, be)
    u1, u2 = _edge_tc(
        ts, td, dist3,
        We1.reshape(1, 16), be1.reshape(1, 16), We2, be2.reshape(1, 16),
        Wn1[0:128], Wn1[128:256], Wn1[256:272], bn1.reshape(1, 128),
        Wc1[0:128], Wc1[128:256], Wc1[256:272], bc1.reshape(1, 128),
        Wc2.reshape(1, 128), e, be)

    zeros_nw = jnp.zeros((n, _WU), jnp.float32)
    s1, s2 = _scatter_sc(u1, u2, dst, zeros_nw, n, e)

    oh, ox = _final_tc(h, x, s1, s2, Wn2, bn2.reshape(1, 128), n, 1000)
    return oh, ox


# speedup vs baseline: 4.9976x; 1.5374x over previous
import functools

import jax
import jax.numpy as jnp
from jax import lax
from jax.experimental import pallas as pl
from jax.experimental.pallas import tpu as pltpu
from jax.experimental.pallas import tpu_sc as plsc

_NC = 2
_NS = 16
_NW = _NC * _NS

_WT = 128
_WU = 128
_CH = 80
_CS = 40


def _sc_mesh():
    return plsc.VectorSubcoreMesh(
        core_axis_name="c", subcore_axis_name="s",
        num_cores=_NC, num_subcores=_NS)


def _gather_sc(t, src3, dst3, n, e):
    epw = e // _NW
    cpt = epw // _CH

    @functools.partial(
        pl.kernel,
        out_type=(
            jax.ShapeDtypeStruct((e, _WT), jnp.float32),
            jax.ShapeDtypeStruct((e, _WT), jnp.float32),
        ),
        mesh=_sc_mesh(),
        scratch_types=[
            pltpu.VMEM((cpt, _CH), jnp.int32),
            pltpu.VMEM((cpt, _CH), jnp.int32),
            pltpu.VMEM((_CH, _WT), jnp.float32),
            pltpu.VMEM((_CH, _WT), jnp.float32),
            pltpu.SemaphoreType.DMA,
            pltpu.SemaphoreType.DMA,
            pltpu.SemaphoreType.DMA,
            pltpu.SemaphoreType.DMA,
        ],
    )
    def k(t_hbm, src_hbm, dst_hbm, ts_hbm, td_hbm,
          idxs_v, idxd_v, rows_s, rows_d, gs, gd, ss, sd):
        wid = lax.axis_index("s") * _NC + lax.axis_index("c")
        base = wid * epw
        pltpu.sync_copy(src_hbm.at[wid], idxs_v)
        pltpu.sync_copy(dst_hbm.at[wid], idxd_v)

        def chunk(i, _):
            off = base + i * _CH
            dgs = pltpu.async_copy(t_hbm.at[idxs_v.at[i]], rows_s, gs)
            dgd = pltpu.async_copy(t_hbm.at[idxd_v.at[i]], rows_d, gd)
            dgs.wait()
            dss = pltpu.async_copy(rows_s, ts_hbm.at[pl.ds(off, _CH)], ss)
            dgd.wait()
            dsd = pltpu.async_copy(rows_d, td_hbm.at[pl.ds(off, _CH)], sd)
            dss.wait()
            dsd.wait()
            return _

        lax.fori_loop(0, cpt, chunk, 0)

    return k(t, src3, dst3)


def _scatter_sc(u1, u2, dst3, zeros_nw, n, e):
    epw = e // _NW
    cpt = epw // _CS
    npairs = cpt // 2
    rpt = (n // _NS) // 8 * 8
    rem = n - _NS * rpt

    @functools.partial(
        pl.kernel,
        out_type=(
            jax.ShapeDtypeStruct((_NC, n, _WU), jnp.float32),
            jax.ShapeDtypeStruct((_NC, n, _WU), jnp.float32),
        ),
        mesh=_sc_mesh(),
        scratch_types=[
            pltpu.VMEM((cpt, _CS), jnp.int32),
            pltpu.VMEM((_CS, _WU), jnp.float32),
            pltpu.VMEM((_CS, _WU), jnp.float32),
            pltpu.VMEM_SHARED((n, _WU), jnp.float32),
            pltpu.SemaphoreType.DMA,
            pltpu.SemaphoreType.DMA,
            pltpu.SemaphoreType.DMA,
            pltpu.SemaphoreType.DMA,
        ],
    )
    def k(u1_hbm, u2_hbm, dst_hbm, z_hbm, s1_hbm, s2_hbm,
          idx_v, rows_a, rows_b, acc, la, lb, aa, ab):
        c = lax.axis_index("c")
        s = lax.axis_index("s")
        wid = s * _NC + c
        base = wid * epw
        mine = pl.ds(s * rpt, rpt)
        tail = pl.ds(_NS * rpt, rem)
        pltpu.sync_copy(dst_hbm.at[wid], idx_v)

        def phase(u_hbm, out_hbm):
            pltpu.sync_copy(z_hbm.at[mine], acc.at[mine])
            @pl.when(s == _NS - 1)
            def _zero_tail():
                pltpu.sync_copy(z_hbm.at[tail], acc.at[tail])
            plsc.subcore_barrier()

            def pair(p, _):
                i0 = 2 * p
                i1 = i0 + 1
                dla = pltpu.async_copy(
                    u_hbm.at[pl.ds(base + i0 * _CS, _CS)], rows_a, la)
                dlb = pltpu.async_copy(
                    u_hbm.at[pl.ds(base + i1 * _CS, _CS)], rows_b, lb)
                dla.wait()
                dsa = pltpu.async_copy(
                    rows_a, acc.at[idx_v.at[i0]], aa, add=True)
                dlb.wait()
                dsb = pltpu.async_copy(
                    rows_b, acc.at[idx_v.at[i1]], ab, add=True)
                dsa.wait()
                dsb.wait()
                return _

            lax.fori_loop(0, npairs, pair, 0)
            plsc.subcore_barrier()
            pltpu.sync_copy(acc.at[mine], out_hbm.at[c].at[mine])
            @pl.when(s == _NS - 1)
            def _write_tail():
                pltpu.sync_copy(acc.at[tail], out_hbm.at[c].at[tail])
            plsc.subcore_barrier()

        phase(u1_hbm, s1_hbm)
        phase(u2_hbm, s2_hbm)

    return k(u1, u2, dst3, zeros_nw)


def _silu(v):
    return v * jax.nn.sigmoid(v)


def _edge_tc(ts, td, dist3, We1r, be1r, We2, be2r, Wn1s, Wn1d, Wn1e, bn1r,
             Wc1s, Wc1d, Wc1e, bc1r, Wc2r, e, be):
    g = e // be

    def unpack(pk):
        pu = jax.lax.bitcast_convert_type(pk, jnp.uint32)
        hi = jax.lax.bitcast_convert_type(
            (pu >> 16).astype(jnp.uint16), jnp.bfloat16)
        lo = jax.lax.bitcast_convert_type(
            pu.astype(jnp.uint16), jnp.bfloat16)
        return jnp.concatenate([hi, lo], axis=1)

    def body(ts_ref, td_ref, d_ref, we1_ref, be1_ref, we2_ref, be2_ref,
             wn1s_ref, wn1d_ref, wn1e_ref, bn1_ref,
             wc1s_ref, wc1d_ref, wc1e_ref, bc1_ref, wc2_ref,
             u1_ref, u2_ref):
        d = d_ref[0, 0, :]
        e1 = d[:, None] * we1_ref[0, :][None, :] + be1_ref[0, :]
        ea = jnp.dot(_silu(e1), we2_ref[...],
                     preferred_element_type=jnp.float32) + be2_ref[0, :]
        hs = unpack(ts_ref[:, 0:64])
        hd = unpack(td_ref[:, 0:64])
        pre_n = (jnp.dot(hs, wn1s_ref[...], preferred_element_type=jnp.float32)
                 + jnp.dot(hd, wn1d_ref[...], preferred_element_type=jnp.float32)
                 + jnp.dot(ea, wn1e_ref[...], preferred_element_type=jnp.float32)
                 + bn1_ref[0, :])
        u1_ref[...] = _silu(pre_n)
        pre_c = (jnp.dot(hs, wc1s_ref[...], preferred_element_type=jnp.float32)
                 + jnp.dot(hd, wc1d_ref[...], preferred_element_type=jnp.float32)
                 + jnp.dot(ea, wc1e_ref[...], preferred_element_type=jnp.float32)
                 + bc1_ref[0, :])
        u = _silu(pre_c)
        cw = jnp.sum(u * wc2_ref[0, :][None, :], axis=1, keepdims=True)
        dvec = ts_ref[:, 64:67] - td_ref[:, 64:67]
        dlen = jnp.maximum(
            jnp.sqrt(jnp.sum(dvec * dvec, axis=1, keepdims=True)), 1e-8)
        cu = cw * (dvec / dlen)
        ones = jnp.ones((be, 1), jnp.float32)
        pad = jnp.zeros((be, _WU - 4), jnp.float32)
        u2_ref[...] = jnp.concatenate([cu, ones, pad], axis=1)

    full = lambda shape: pl.BlockSpec(shape, lambda i: (0,) * len(shape))
    return pl.pallas_call(
        body,
        grid=(g,),
        in_specs=[
            pl.BlockSpec((be, _WT), lambda i: (i, 0)),
            pl.BlockSpec((be, _WT), lambda i: (i, 0)),
            pl.BlockSpec((1, 1, be), lambda i: (i, 0, 0)),
            full((1, 16)), full((1, 16)), full((16, 16)), full((1, 16)),
            full((128, 128)), full((128, 128)), full((16, 128)), full((1, 128)),
            full((128, 128)), full((128, 128)), full((16, 128)), full((1, 128)),
            full((1, 128)),
        ],
        out_specs=[
            pl.BlockSpec((be, _WU), lambda i: (i, 0)),
            pl.BlockSpec((be, _WU), lambda i: (i, 0)),
        ],
        out_shape=[
            jax.ShapeDtypeStruct((e, _WU), jnp.float32),
            jax.ShapeDtypeStruct((e, _WU), jnp.float32),
        ],
    )(ts, td, dist3, We1r, be1r, We2, be2r, Wn1s, Wn1d, Wn1e, bn1r,
      Wc1s, Wc1d, Wc1e, bc1r, Wc2r)


def _final_tc(h, x3, s1, s2, Wn2, bn2r, n, bn):
    g = n // bn

    def body(h_ref, x_ref, s1_ref, s2_ref, wn2_ref, bn2_ref, oh_ref, ox_ref):
        hidden = s1_ref[0] + s1_ref[1]
        s2 = s2_ref[0] + s2_ref[1]
        deg = s2[:, 3:4]
        xa = s2[:, 0:3]
        oh_ref[...] = (h_ref[...]
                       + jnp.dot(hidden, wn2_ref[...],
                                 preferred_element_type=jnp.float32)
                       + deg * bn2_ref[0, :])
        ox_ref[...] = x_ref[...] + xa

    return pl.pallas_call(
        body,
        grid=(g,),
        in_specs=[
            pl.BlockSpec((bn, 128), lambda i: (i, 0)),
            pl.BlockSpec((bn, 3), lambda i: (i, 0)),
            pl.BlockSpec((_NC, bn, _WU), lambda i: (0, i, 0)),
            pl.BlockSpec((_NC, bn, _WU), lambda i: (0, i, 0)),
            pl.BlockSpec((128, 128), lambda i: (0, 0)),
            pl.BlockSpec((1, 128), lambda i: (0, 0)),
        ],
        out_specs=[
            pl.BlockSpec((bn, 128), lambda i: (i, 0)),
            pl.BlockSpec((bn, 3), lambda i: (i, 0)),
        ],
        out_shape=[
            jax.ShapeDtypeStruct((n, 128), jnp.float32),
            jax.ShapeDtypeStruct((n, 3), jnp.float32),
        ],
    )(h, x3, s1, s2, Wn2, bn2r)


@jax.jit
def kernel(h, x, edge_index, edge_dist, We1, be1, We2, be2, Wn1, bn1, Wn2,
           bn2, Wc1, bc1, Wc2):
    n, nd = h.shape
    e = edge_dist.shape[0]
    epw = e // _NW
    assert nd == 128 and e % _NW == 0 and epw % _CH == 0
    assert epw % (2 * _CS) == 0 and n % _NS == 0

    src = edge_index[0]
    dst = edge_index[1]
    src3g = src.reshape(_NW, epw // _CH, _CH)
    dst3g = dst.reshape(_NW, epw // _CH, _CH)
    dst3s = dst.reshape(_NW, epw // _CS, _CS)

    hb = h.astype(jnp.bfloat16).reshape(n, 64, 2)
    hi16 = jax.lax.bitcast_convert_type(hb[:, :, 0], jnp.uint16)
    lo16 = jax.lax.bitcast_convert_type(hb[:, :, 1], jnp.uint16)
    packed = jax.lax.bitcast_convert_type(
        (hi16.astype(jnp.uint32) << 16) | lo16.astype(jnp.uint32),
        jnp.float32)
    t = jnp.concatenate(
        [packed, x, jnp.zeros((n, _WT - 67), jnp.float32)], axis=1)

    ts, td = _gather_sc(t, src3g, dst3g, n, e)

    perm = jnp.concatenate(
        [jnp.arange(0, 128, 2), jnp.arange(1, 128, 2)])
    bf = jnp.bfloat16
    be = 2560
    dist3 = edge_dist.reshape(e // be, 1, be)
    u1, u2 = _edge_tc(
        ts, td, dist3,
        We1.reshape(1, 16), be1.reshape(1, 16), We2, be2.reshape(1, 16),
        Wn1[0:128][perm].astype(bf), Wn1[128:256][perm].astype(bf),
        Wn1[256:272], bn1.reshape(1, 128),
        Wc1[0:128][perm].astype(bf), Wc1[128:256][perm].astype(bf),
        Wc1[256:272], bc1.reshape(1, 128),
        Wc2.reshape(1, 128), e, be)

    zeros_nw = jnp.zeros((n, _WU), jnp.float32)
    s1, s2 = _scatter_sc(u1, u2, dst3s, zeros_nw, n, e)

    oh, ox = _final_tc(h, x, s1, s2, Wn2, bn2.reshape(1, 128), n, 1000)
    return oh, ox


# speedup vs baseline: 5.5244x; 1.1054x over previous
import functools

import jax
import jax.numpy as jnp
from jax import lax
from jax.experimental import pallas as pl
from jax.experimental.pallas import tpu as pltpu
from jax.experimental.pallas import tpu_sc as plsc

_NC = 2
_NS = 16
_NW = _NC * _NS

_K = 2
_WT = 128
_WU = 128
_CH = 40
_CS = 40


def _sc_mesh():
    return plsc.VectorSubcoreMesh(
        core_axis_name="c", subcore_axis_name="s",
        num_cores=_NC, num_subcores=_NS)


def _gather_sc(t, src3, dst3, n, e):
    epw = e // _NW
    cpt = epw // _CH

    @functools.partial(
        pl.kernel,
        out_type=(
            jax.ShapeDtypeStruct((e, _WT), jnp.float32),
            jax.ShapeDtypeStruct((e, _WT), jnp.float32),
        ),
        mesh=_sc_mesh(),
        scratch_types=[
            pltpu.VMEM((cpt, _CH), jnp.int32),
            pltpu.VMEM((cpt, _CH), jnp.int32),
            pltpu.VMEM((_CH, _WT), jnp.float32),
            pltpu.VMEM((_CH, _WT), jnp.float32),
            pltpu.SemaphoreType.DMA,
            pltpu.SemaphoreType.DMA,
            pltpu.SemaphoreType.DMA,
            pltpu.SemaphoreType.DMA,
        ],
    )
    def k(t_hbm, src_hbm, dst_hbm, ts_hbm, td_hbm,
          idxs_v, idxd_v, rows_s, rows_d, gs, gd, ss, sd):
        wid = lax.axis_index("s") * _NC + lax.axis_index("c")
        base = wid * epw
        pltpu.sync_copy(src_hbm.at[wid], idxs_v)
        pltpu.sync_copy(dst_hbm.at[wid], idxd_v)

        def chunk(i, _):
            off = base + i * _CH
            dgs = pltpu.async_copy(t_hbm.at[idxs_v.at[i]], rows_s, gs)
            dgd = pltpu.async_copy(t_hbm.at[idxd_v.at[i]], rows_d, gd)
            dgs.wait()
            dss = pltpu.async_copy(rows_s, ts_hbm.at[pl.ds(off, _CH)], ss)
            dgd.wait()
            dsd = pltpu.async_copy(rows_d, td_hbm.at[pl.ds(off, _CH)], sd)
            dss.wait()
            dsd.wait()
            return _

        lax.fori_loop(0, cpt, chunk, 0)

    return k(t, src3, dst3)


def _scatter_sc(u1, u2, dst3, zeros_nw, n, e):
    epw = e // _NW
    cpt = epw // _CS
    npairs = cpt // 2
    rpt = (n // _NS) // 8 * 8
    rem = n - _NS * rpt

    @functools.partial(
        pl.kernel,
        out_type=(
            jax.ShapeDtypeStruct((_NC, n, _WU), jnp.float32),
            jax.ShapeDtypeStruct((_NC, n, _WU), jnp.float32),
        ),
        mesh=_sc_mesh(),
        scratch_types=[
            pltpu.VMEM((cpt, _CS), jnp.int32),
            pltpu.VMEM((_CS, _WU), jnp.float32),
            pltpu.VMEM((_CS, _WU), jnp.float32),
            pltpu.VMEM_SHARED((n, _WU), jnp.float32),
            pltpu.SemaphoreType.DMA,
            pltpu.SemaphoreType.DMA,
            pltpu.SemaphoreType.DMA,
            pltpu.SemaphoreType.DMA,
        ],
    )
    def k(u1_hbm, u2_hbm, dst_hbm, z_hbm, s1_hbm, s2_hbm,
          idx_v, rows_a, rows_b, acc, la, lb, aa, ab):
        c = lax.axis_index("c")
        s = lax.axis_index("s")
        wid = s * _NC + c
        base = wid * epw
        mine = pl.ds(s * rpt, rpt)
        tail = pl.ds(_NS * rpt, rem)
        pltpu.sync_copy(dst_hbm.at[wid], idx_v)

        def phase(u_hbm, out_hbm):
            pltpu.sync_copy(z_hbm.at[mine], acc.at[mine])
            @pl.when(s == _NS - 1)
            def _zero_tail():
                pltpu.sync_copy(z_hbm.at[tail], acc.at[tail])
            plsc.subcore_barrier()

            def pair(p, _):
                i0 = 2 * p
                i1 = i0 + 1
                dla = pltpu.async_copy(
                    u_hbm.at[pl.ds(base + i0 * _CS, _CS)], rows_a, la)
                dlb = pltpu.async_copy(
                    u_hbm.at[pl.ds(base + i1 * _CS, _CS)], rows_b, lb)
                dla.wait()
                dsa = pltpu.async_copy(
                    rows_a, acc.at[idx_v.at[i0]], aa, add=True)
                dlb.wait()
                dsb = pltpu.async_copy(
                    rows_b, acc.at[idx_v.at[i1]], ab, add=True)
                dsa.wait()
                dsb.wait()
                return _

            lax.fori_loop(0, npairs, pair, 0)
            if cpt > 2 * npairs:
                i_last = cpt - 1
                pltpu.async_copy(
                    u_hbm.at[pl.ds(base + i_last * _CS, _CS)],
                    rows_a, la).wait()
                pltpu.async_copy(
                    rows_a, acc.at[idx_v.at[i_last]], aa, add=True).wait()
            plsc.subcore_barrier()
            pltpu.sync_copy(acc.at[mine], out_hbm.at[c].at[mine])
            @pl.when(s == _NS - 1)
            def _write_tail():
                pltpu.sync_copy(acc.at[tail], out_hbm.at[c].at[tail])
            plsc.subcore_barrier()

        phase(u1_hbm, s1_hbm)
        phase(u2_hbm, s2_hbm)

    return k(u1, u2, dst3, zeros_nw)


def _silu(v):
    return v * jax.nn.sigmoid(v)


def _edge_tc(ts, td, dist3, We1r, be1r, We2, be2r, Wn1s, Wn1d, Wn1e, bn1r,
             Wc1s, Wc1d, Wc1e, bc1r, Wc2r, e, be):
    g = e // be

    def unpack(pk):
        pu = jax.lax.bitcast_convert_type(pk, jnp.uint32)
        hi = jax.lax.bitcast_convert_type(
            (pu >> 16).astype(jnp.uint16), jnp.bfloat16)
        lo = jax.lax.bitcast_convert_type(
            pu.astype(jnp.uint16), jnp.bfloat16)
        return jnp.concatenate([hi, lo], axis=1)

    def body(ts_ref, td_ref, d_ref, we1_ref, be1_ref, we2_ref, be2_ref,
             wn1s_ref, wn1d_ref, wn1e_ref, bn1_ref,
             wc1s_ref, wc1d_ref, wc1e_ref, bc1_ref, wc2_ref,
             u1_ref, u2_ref):
        d = d_ref[0, 0, :]
        e1 = d[:, None] * we1_ref[0, :][None, :] + be1_ref[0, :]
        ea = jnp.dot(_silu(e1), we2_ref[...],
                     preferred_element_type=jnp.float32) + be2_ref[0, :]
        hs = unpack(ts_ref[:, 0:64])
        hd = unpack(td_ref[:, 0:64])
        pre_n = (jnp.dot(hs, wn1s_ref[...], preferred_element_type=jnp.float32)
                 + jnp.dot(hd, wn1d_ref[...], preferred_element_type=jnp.float32)
                 + jnp.dot(ea, wn1e_ref[...], preferred_element_type=jnp.float32)
                 + bn1_ref[0, :])
        u1_ref[...] = _silu(pre_n)
        pre_c = (jnp.dot(hs, wc1s_ref[...], preferred_element_type=jnp.float32)
                 + jnp.dot(hd, wc1d_ref[...], preferred_element_type=jnp.float32)
                 + jnp.dot(ea, wc1e_ref[...], preferred_element_type=jnp.float32)
                 + bc1_ref[0, :])
        u = _silu(pre_c)
        cw = jnp.sum(u * wc2_ref[0, :][None, :], axis=1, keepdims=True)
        dvec = ts_ref[:, 64:67] - td_ref[:, 64:67]
        dlen = jnp.maximum(
            jnp.sqrt(jnp.sum(dvec * dvec, axis=1, keepdims=True)), 1e-8)
        cu = cw * (dvec / dlen)
        ones = jnp.ones((be, 1), jnp.float32)
        pad = jnp.zeros((be, _WU - 4), jnp.float32)
        u2_ref[...] = jnp.concatenate([cu, ones, pad], axis=1)

    full = lambda shape: pl.BlockSpec(shape, lambda i: (0,) * len(shape))
    return pl.pallas_call(
        body,
        grid=(g,),
        in_specs=[
            pl.BlockSpec((be, _WT), lambda i: (i, 0)),
            pl.BlockSpec((be, _WT), lambda i: (i, 0)),
            pl.BlockSpec((1, 1, be), lambda i: (i, 0, 0)),
            full((1, 16)), full((1, 16)), full((16, 16)), full((1, 16)),
            full((128, 128)), full((128, 128)), full((16, 128)), full((1, 128)),
            full((128, 128)), full((128, 128)), full((16, 128)), full((1, 128)),
            full((1, 128)),
        ],
        out_specs=[
            pl.BlockSpec((be, _WU), lambda i: (i, 0)),
            pl.BlockSpec((be, _WU), lambda i: (i, 0)),
        ],
        out_shape=[
            jax.ShapeDtypeStruct((e, _WU), jnp.float32),
            jax.ShapeDtypeStruct((e, _WU), jnp.float32),
        ],
    )(ts, td, dist3, We1r, be1r, We2, be2r, Wn1s, Wn1d, Wn1e, bn1r,
      Wc1s, Wc1d, Wc1e, bc1r, Wc2r)


def _final_tc(h, x3, s1s, s2s, Wn2, bn2r, n, bn):
    g = n // bn
    np_ = len(s1s)

    def body(*refs):
        h_ref, x_ref = refs[0], refs[1]
        s1_refs = refs[2:2 + np_]
        s2_refs = refs[2 + np_:2 + 2 * np_]
        wn2_ref, bn2_ref, oh_ref, ox_ref = refs[2 + 2 * np_:]
        hidden = s1_refs[0][0] + s1_refs[0][1]
        s2 = s2_refs[0][0] + s2_refs[0][1]
        for r in s1_refs[1:]:
            hidden = hidden + r[0] + r[1]
        for r in s2_refs[1:]:
            s2 = s2 + r[0] + r[1]
        deg = s2[:, 3:4]
        xa = s2[:, 0:3]
        oh_ref[...] = (h_ref[...]
                       + jnp.dot(hidden, wn2_ref[...],
                                 preferred_element_type=jnp.float32)
                       + deg * bn2_ref[0, :])
        ox_ref[...] = x_ref[...] + xa

    part_spec = pl.BlockSpec((_NC, bn, _WU), lambda i: (0, i, 0))
    return pl.pallas_call(
        body,
        grid=(g,),
        in_specs=[
            pl.BlockSpec((bn, 128), lambda i: (i, 0)),
            pl.BlockSpec((bn, 3), lambda i: (i, 0)),
        ] + [part_spec] * (2 * np_) + [
            pl.BlockSpec((128, 128), lambda i: (0, 0)),
            pl.BlockSpec((1, 128), lambda i: (0, 0)),
        ],
        out_specs=[
            pl.BlockSpec((bn, 128), lambda i: (i, 0)),
            pl.BlockSpec((bn, 3), lambda i: (i, 0)),
        ],
        out_shape=[
            jax.ShapeDtypeStruct((n, 128), jnp.float32),
            jax.ShapeDtypeStruct((n, 3), jnp.float32),
        ],
    )(h, x3, *s1s, *s2s, Wn2, bn2r)


@jax.jit
def kernel(h, x, edge_index, edge_dist, We1, be1, We2, be2, Wn1, bn1, Wn2,
           bn2, Wc1, bc1, Wc2):
    n, nd = h.shape
    e = edge_dist.shape[0]
    e2 = e // _K
    epw = e2 // _NW
    assert nd == 128 and e2 % _NW == 0 and epw % _CH == 0
    assert epw % _CS == 0 and n % _NS == 0

    src = edge_index[0].reshape(_K, e2)
    dst = edge_index[1].reshape(_K, e2)
    src3g = src.reshape(_K, _NW, epw // _CH, _CH)
    dst3g = dst.reshape(_K, _NW, epw // _CH, _CH)
    dst3s = dst.reshape(_K, _NW, epw // _CS, _CS)

    hb = h.astype(jnp.bfloat16).reshape(n, 64, 2)
    hi16 = jax.lax.bitcast_convert_type(hb[:, :, 0], jnp.uint16)
    lo16 = jax.lax.bitcast_convert_type(hb[:, :, 1], jnp.uint16)
    packed = jax.lax.bitcast_convert_type(
        (hi16.astype(jnp.uint32) << 16) | lo16.astype(jnp.uint32),
        jnp.float32)
    t = jnp.concatenate(
        [packed, x, jnp.zeros((n, _WT - 67), jnp.float32)], axis=1)

    perm = jnp.concatenate(
        [jnp.arange(0, 128, 2), jnp.arange(1, 128, 2)])
    bf = jnp.bfloat16
    be = 3200
    dist3 = edge_dist.reshape(_K, e2 // be, 1, be)
    zeros_nw = jnp.zeros((n, _WU), jnp.float32)

    def chain(idx, tok):
        return idx + (tok * 0.0).astype(jnp.int32)

    gathered = []
    tok = None
    for k in range(_K):
        s3 = src3g[k] if tok is None else chain(src3g[k], tok)
        ts, td = _gather_sc(t, s3, dst3g[k], n, e2)
        tok = ts[0, 0]
        gathered.append((ts, td))
    parts = []
    for k in range(_K):
        ts, td = gathered[k]
        u1, u2 = _edge_tc(
            ts, td, dist3[k],
            We1.reshape(1, 16), be1.reshape(1, 16), We2, be2.reshape(1, 16),
            Wn1[0:128][perm].astype(bf), Wn1[128:256][perm].astype(bf),
            Wn1[256:272], bn1.reshape(1, 128),
            Wc1[0:128][perm].astype(bf), Wc1[128:256][perm].astype(bf),
            Wc1[256:272], bc1.reshape(1, 128),
            Wc2.reshape(1, 128), e2, be)
        p = _scatter_sc(u1, u2, chain(dst3s[k], tok), zeros_nw, n, e2)
        tok = p[0][0, 0, 0]
        parts.append(p)

    oh, ox = _final_tc(h, x, [p[0] for p in parts], [p[1] for p in parts],
                       Wn2, bn2.reshape(1, 128), n, 1000)
    return oh, ox


# speedup vs baseline: 6.5079x; 1.1780x over previous
import functools

import jax
import jax.numpy as jnp
from jax import lax
from jax.experimental import pallas as pl
from jax.experimental.pallas import tpu as pltpu
from jax.experimental.pallas import tpu_sc as plsc

_NC = 2
_NS = 16
_NW = _NC * _NS

_K = 2
_WT = 128
_WU = 128
_CH = 80
_CS = 40
_NB = 4


def _sc_mesh():
    return plsc.VectorSubcoreMesh(
        core_axis_name="c", subcore_axis_name="s",
        num_cores=_NC, num_subcores=_NS)


def _gather_sc(t, src2, dst2, n, e):
    epw = e // _NW
    nfull = epw // _CH
    tl = epw - nfull * _CH

    @functools.partial(
        pl.kernel,
        out_type=(
            jax.ShapeDtypeStruct((e, _WT), jnp.float32),
            jax.ShapeDtypeStruct((e, _WT), jnp.float32),
        ),
        mesh=_sc_mesh(),
        scratch_types=[
            pltpu.VMEM((epw,), jnp.int32),
            pltpu.VMEM((epw,), jnp.int32),
            pltpu.VMEM((_CH, _WT), jnp.float32),
            pltpu.VMEM((_CH, _WT), jnp.float32),
            pltpu.SemaphoreType.DMA,
            pltpu.SemaphoreType.DMA,
            pltpu.SemaphoreType.DMA,
            pltpu.SemaphoreType.DMA,
        ],
    )
    def k(t_hbm, src_hbm, dst_hbm, ts_hbm, td_hbm,
          idxs_v, idxd_v, rows_s, rows_d, gs, gd, ss, sd):
        wid = lax.axis_index("s") * _NC + lax.axis_index("c")
        base = wid * epw
        pltpu.sync_copy(src_hbm.at[wid], idxs_v)
        pltpu.sync_copy(dst_hbm.at[wid], idxd_v)

        def step(i, sz):
            off = base + i * _CH
            iv = pl.ds(i * _CH, sz)
            rs = rows_s.at[pl.ds(0, sz)]
            rd = rows_d.at[pl.ds(0, sz)]
            dgs = pltpu.async_copy(t_hbm.at[idxs_v.at[iv]], rs, gs)
            dgd = pltpu.async_copy(t_hbm.at[idxd_v.at[iv]], rd, gd)
            dgs.wait()
            dss = pltpu.async_copy(rs, ts_hbm.at[pl.ds(off, sz)], ss)
            dgd.wait()
            dsd = pltpu.async_copy(rd, td_hbm.at[pl.ds(off, sz)], sd)
            dss.wait()
            dsd.wait()

        def chunk(i, _):
            step(i, _CH)
            return _

        lax.fori_loop(0, nfull, chunk, 0)
        if tl:
            step(nfull, tl)

    return k(t, src2, dst2)


def _scatter_sc(u1, u2, dst3, zeros_nw, n, e):
    epw = e // _NW
    cpt = epw // _CS
    ngrp = cpt // _NB
    nleft = cpt - ngrp * _NB
    rpt = (n // _NS) // 8 * 8
    rem = n - _NS * rpt

    @functools.partial(
        pl.kernel,
        out_type=(
            jax.ShapeDtypeStruct((_NC, n, _WU), jnp.float32),
            jax.ShapeDtypeStruct((_NC, n, _WU), jnp.float32),
        ),
        mesh=_sc_mesh(),
        scratch_types=[
            pltpu.VMEM((cpt, _CS), jnp.int32),
        ] + [pltpu.VMEM((_CS, _WU), jnp.float32)] * _NB + [
            pltpu.VMEM_SHARED((n, _WU), jnp.float32),
        ] + [pltpu.SemaphoreType.DMA] * (2 * _NB),
    )
    def k(u1_hbm, u2_hbm, dst_hbm, z_hbm, s1_hbm, s2_hbm,
          idx_v, *rest):
        rows = rest[:_NB]
        acc = rest[_NB]
        lsem = rest[_NB + 1:_NB + 1 + _NB]
        asem = rest[_NB + 1 + _NB:]
        c = lax.axis_index("c")
        s = lax.axis_index("s")
        wid = s * _NC + c
        base = wid * epw
        mine = pl.ds(s * rpt, rpt)
        tail = pl.ds(_NS * rpt, rem)
        pltpu.sync_copy(dst_hbm.at[wid], idx_v)

        def phase(u_hbm, out_hbm):
            pltpu.sync_copy(z_hbm.at[mine], acc.at[mine])
            @pl.when(s == _NS - 1)
            def _zero_tail():
                pltpu.sync_copy(z_hbm.at[tail], acc.at[tail])
            plsc.subcore_barrier()

            def burst(i0, nb):
                loads = [pltpu.async_copy(
                    u_hbm.at[pl.ds(base + (i0 + b) * _CS, _CS)],
                    rows[b], lsem[b]) for b in range(nb)]
                adds = []
                for b in range(nb):
                    loads[b].wait()
                    adds.append(pltpu.async_copy(
                        rows[b], acc.at[idx_v.at[i0 + b]], asem[b],
                        add=True))
                for d in adds:
                    d.wait()

            def grp(p, _):
                burst(p * _NB, _NB)
                return _

            lax.fori_loop(0, ngrp, grp, 0)
            if nleft:
                burst(ngrp * _NB, nleft)
            plsc.subcore_barrier()
            pltpu.sync_copy(acc.at[mine], out_hbm.at[c].at[mine])
            @pl.when(s == _NS - 1)
            def _write_tail():
                pltpu.sync_copy(acc.at[tail], out_hbm.at[c].at[tail])
            plsc.subcore_barrier()

        phase(u1_hbm, s1_hbm)
        phase(u2_hbm, s2_hbm)

    return k(u1, u2, dst3, zeros_nw)


def _silu(v):
    return v * jax.nn.sigmoid(v)


def _edge_tc(ts, td, dist3, We1r, be1r, We2, be2r, Wn1s, Wn1d, Wn1e, bn1r,
             Wc1s, Wc1d, Wc1e, bc1r, Wc2r, e, be):
    g = e // be

    def unpack(pk):
        pu = jax.lax.bitcast_convert_type(pk, jnp.uint32)
        hi = jax.lax.bitcast_convert_type(
            (pu >> 16).astype(jnp.uint16), jnp.bfloat16)
        lo = jax.lax.bitcast_convert_type(
            pu.astype(jnp.uint16), jnp.bfloat16)
        return jnp.concatenate([hi, lo], axis=1)

    def body(ts_ref, td_ref, d_ref, we1_ref, be1_ref, we2_ref, be2_ref,
             wn1s_ref, wn1d_ref, wn1e_ref, bn1_ref,
             wc1s_ref, wc1d_ref, wc1e_ref, bc1_ref, wc2_ref,
             u1_ref, u2_ref):
        d = d_ref[0, 0, :]
        e1 = d[:, None] * we1_ref[0, :][None, :] + be1_ref[0, :]
        ea = jnp.dot(_silu(e1), we2_ref[...],
                     preferred_element_type=jnp.float32) + be2_ref[0, :]
        hs = unpack(ts_ref[:, 0:64])
        hd = unpack(td_ref[:, 0:64])
        pre_n = (jnp.dot(hs, wn1s_ref[...], preferred_element_type=jnp.float32)
                 + jnp.dot(hd, wn1d_ref[...], preferred_element_type=jnp.float32)
                 + jnp.dot(ea, wn1e_ref[...], preferred_element_type=jnp.float32)
                 + bn1_ref[0, :])
        u1_ref[...] = _silu(pre_n)
        pre_c = (jnp.dot(hs, wc1s_ref[...], preferred_element_type=jnp.float32)
                 + jnp.dot(hd, wc1d_ref[...], preferred_element_type=jnp.float32)
                 + jnp.dot(ea, wc1e_ref[...], preferred_element_type=jnp.float32)
                 + bc1_ref[0, :])
        u = _silu(pre_c)
        cw = jnp.sum(u * wc2_ref[0, :][None, :], axis=1, keepdims=True)
        dvec = ts_ref[:, 64:67] - td_ref[:, 64:67]
        dlen = jnp.maximum(
            jnp.sqrt(jnp.sum(dvec * dvec, axis=1, keepdims=True)), 1e-8)
        cu = cw * (dvec / dlen)
        ones = jnp.ones((be, 1), jnp.float32)
        pad = jnp.zeros((be, _WU - 4), jnp.float32)
        u2_ref[...] = jnp.concatenate([cu, ones, pad], axis=1)

    full = lambda shape: pl.BlockSpec(shape, lambda i: (0,) * len(shape))
    return pl.pallas_call(
        body,
        grid=(g,),
        in_specs=[
            pl.BlockSpec((be, _WT), lambda i: (i, 0)),
            pl.BlockSpec((be, _WT), lambda i: (i, 0)),
            pl.BlockSpec((1, 1, be), lambda i: (i, 0, 0)),
            full((1, 16)), full((1, 16)), full((16, 16)), full((1, 16)),
            full((128, 128)), full((128, 128)), full((16, 128)), full((1, 128)),
            full((128, 128)), full((128, 128)), full((16, 128)), full((1, 128)),
            full((1, 128)),
        ],
        out_specs=[
            pl.BlockSpec((be, _WU), lambda i: (i, 0)),
            pl.BlockSpec((be, _WU), lambda i: (i, 0)),
        ],
        out_shape=[
            jax.ShapeDtypeStruct((e, _WU), jnp.float32),
            jax.ShapeDtypeStruct((e, _WU), jnp.float32),
        ],
    )(ts, td, dist3, We1r, be1r, We2, be2r, Wn1s, Wn1d, Wn1e, bn1r,
      Wc1s, Wc1d, Wc1e, bc1r, Wc2r)


def _final_tc(h, x3, s1s, s2s, Wn2, bn2r, n, bn):
    g = n // bn
    np_ = len(s1s)

    def body(*refs):
        h_ref, x_ref = refs[0], refs[1]
        s1_refs = refs[2:2 + np_]
        s2_refs = refs[2 + np_:2 + 2 * np_]
        wn2_ref, bn2_ref, oh_ref, ox_ref = refs[2 + 2 * np_:]
        hidden = s1_refs[0][0] + s1_refs[0][1]
        s2 = s2_refs[0][0] + s2_refs[0][1]
        for r in s1_refs[1:]:
            hidden = hidden + r[0] + r[1]
        for r in s2_refs[1:]:
            s2 = s2 + r[0] + r[1]
        deg = s2[:, 3:4]
        xa = s2[:, 0:3]
        oh_ref[...] = (h_ref[...]
                       + jnp.dot(hidden, wn2_ref[...],
                                 preferred_element_type=jnp.float32)
                       + deg * bn2_ref[0, :])
        ox_ref[...] = x_ref[...] + xa

    part_spec = pl.BlockSpec((_NC, bn, _WU), lambda i: (0, i, 0))
    return pl.pallas_call(
        body,
        grid=(g,),
        in_specs=[
            pl.BlockSpec((bn, 128), lambda i: (i, 0)),
            pl.BlockSpec((bn, 3), lambda i: (i, 0)),
        ] + [part_spec] * (2 * np_) + [
            pl.BlockSpec((128, 128), lambda i: (0, 0)),
            pl.BlockSpec((1, 128), lambda i: (0, 0)),
        ],
        out_specs=[
            pl.BlockSpec((bn, 128), lambda i: (i, 0)),
            pl.BlockSpec((bn, 3), lambda i: (i, 0)),
        ],
        out_shape=[
            jax.ShapeDtypeStruct((n, 128), jnp.float32),
            jax.ShapeDtypeStruct((n, 3), jnp.float32),
        ],
    )(h, x3, *s1s, *s2s, Wn2, bn2r)


@jax.jit
def kernel(h, x, edge_index, edge_dist, We1, be1, We2, be2, Wn1, bn1, Wn2,
           bn2, Wc1, bc1, Wc2):
    n, nd = h.shape
    e = edge_dist.shape[0]
    e2 = e // _K
    epw = e2 // _NW
    assert nd == 128 and e2 % _NW == 0 and epw % 8 == 0
    assert epw % _CS == 0 and n % _NS == 0

    src = edge_index[0].reshape(_K, e2)
    dst = edge_index[1].reshape(_K, e2)
    src3g = src.reshape(_K, _NW, epw)
    dst3g = dst.reshape(_K, _NW, epw)
    dst3s = dst.reshape(_K, _NW, epw // _CS, _CS)

    hb = h.astype(jnp.bfloat16).reshape(n, 64, 2)
    hi16 = jax.lax.bitcast_convert_type(hb[:, :, 0], jnp.uint16)
    lo16 = jax.lax.bitcast_convert_type(hb[:, :, 1], jnp.uint16)
    packed = jax.lax.bitcast_convert_type(
        (hi16.astype(jnp.uint32) << 16) | lo16.astype(jnp.uint32),
        jnp.float32)
    t = jnp.concatenate(
        [packed, x, jnp.zeros((n, _WT - 67), jnp.float32)], axis=1)

    perm = jnp.concatenate(
        [jnp.arange(0, 128, 2), jnp.arange(1, 128, 2)])
    bf = jnp.bfloat16
    be = 3200
    dist3 = edge_dist.reshape(_K, e2 // be, 1, be)
    zeros_nw = jnp.zeros((n, _WU), jnp.float32)

    def chain(idx, tok):
        return idx + (tok * 0.0).astype(jnp.int32)

    gathered = []
    tok = None
    for k in range(_K):
        s3 = src3g[k] if tok is None else chain(src3g[k], tok)
        ts, td = _gather_sc(t, s3, dst3g[k], n, e2)
        tok = ts[0, 0]
        gathered.append((ts, td))
    parts = []
    for k in range(_K):
        ts, td = gathered[k]
        u1, u2 = _edge_tc(
            ts, td, dist3[k],
            We1.reshape(1, 16), be1.reshape(1, 16), We2, be2.reshape(1, 16),
            Wn1[0:128][perm].astype(bf), Wn1[128:256][perm].astype(bf),
            Wn1[256:272], bn1.reshape(1, 128),
            Wc1[0:128][perm].astype(bf), Wc1[128:256][perm].astype(bf),
            Wc1[256:272], bc1.reshape(1, 128),
            Wc2.reshape(1, 128), e2, be)
        p = _scatter_sc(u1, u2, chain(dst3s[k], tok), zeros_nw, n, e2)
        tok = p[0][0, 0, 0]
        parts.append(p)

    oh, ox = _final_tc(h, x, [p[0] for p in parts], [p[1] for p in parts],
                       Wn2, bn2.reshape(1, 128), n, 1000)
    return oh, ox


# speedup vs baseline: 6.6026x; 1.0146x over previous
import functools

import jax
import jax.numpy as jnp
from jax import lax
from jax.experimental import pallas as pl
from jax.experimental.pallas import tpu as pltpu
from jax.experimental.pallas import tpu_sc as plsc

_NC = 2
_NS = 16
_NW = _NC * _NS

_K = 2
_WT = 128
_WU = 128
_CH = 80
_CS = 40
_NB = 4


def _sc_mesh():
    return plsc.VectorSubcoreMesh(
        core_axis_name="c", subcore_axis_name="s",
        num_cores=_NC, num_subcores=_NS)


def _gather_sc(t, src2, dst2, n, e):
    epw = e // _NW
    nfull = epw // _CH
    tl = epw - nfull * _CH

    @functools.partial(
        pl.kernel,
        out_type=(
            jax.ShapeDtypeStruct((e, _WT), jnp.float32),
            jax.ShapeDtypeStruct((e, _WT), jnp.float32),
        ),
        mesh=_sc_mesh(),
        scratch_types=[
            pltpu.VMEM((epw,), jnp.int32),
            pltpu.VMEM((epw,), jnp.int32),
        ] + [pltpu.VMEM((_CH, _WT), jnp.float32)] * _NB
          + [pltpu.SemaphoreType.DMA] * (2 * _NB),
    )
    def k(t_hbm, src_hbm, dst_hbm, ts_hbm, td_hbm, idxs_v, idxd_v, *rest):
        rows = rest[:_NB]
        gsem = rest[_NB:2 * _NB]
        ssem = rest[2 * _NB:]
        wid = lax.axis_index("s") * _NC + lax.axis_index("c")
        base = wid * epw
        pltpu.sync_copy(src_hbm.at[wid], idxs_v)
        pltpu.sync_copy(dst_hbm.at[wid], idxd_v)

        def burst(i0, nch, sz):
            jobs = [(side, i0 + j)
                    for j in range(nch) for side in (0, 1)]
            gets = []
            for b, (side, i) in enumerate(jobs):
                ih = idxs_v if side == 0 else idxd_v
                iv = pl.ds(i * _CH, sz)
                gets.append(pltpu.async_copy(
                    t_hbm.at[ih.at[iv]], rows[b].at[pl.ds(0, sz)], gsem[b]))
            puts = []
            for b, (side, i) in enumerate(jobs):
                oh = ts_hbm if side == 0 else td_hbm
                gets[b].wait()
                puts.append(pltpu.async_copy(
                    rows[b].at[pl.ds(0, sz)],
                    oh.at[pl.ds(base + i * _CH, sz)], ssem[b]))
            for d in puts:
                d.wait()

        npc = _NB // 2
        def grp(p, _):
            burst(p * npc, npc, _CH)
            return _

        lax.fori_loop(0, nfull // npc, grp, 0)
        for i in range(nfull - nfull % npc, nfull):
            burst(i, 1, _CH)
        if tl:
            burst(nfull, 1, tl)

    return k(t, src2, dst2)


def _scatter_sc(u1, u2, dst3, zeros_nw, n, e):
    epw = e // _NW
    cpt = epw // _CS
    ngrp = cpt // _NB
    nleft = cpt - ngrp * _NB
    rpt = (n // _NS) // 8 * 8
    rem = n - _NS * rpt

    @functools.partial(
        pl.kernel,
        out_type=(
            jax.ShapeDtypeStruct((_NC, n, _WU), jnp.float32),
            jax.ShapeDtypeStruct((_NC, n, _WU), jnp.float32),
        ),
        mesh=_sc_mesh(),
        scratch_types=[
            pltpu.VMEM((cpt, _CS), jnp.int32),
        ] + [pltpu.VMEM((_CS, _WU), jnp.float32)] * _NB + [
            pltpu.VMEM_SHARED((n, _WU), jnp.float32),
        ] + [pltpu.SemaphoreType.DMA] * (2 * _NB),
    )
    def k(u1_hbm, u2_hbm, dst_hbm, z_hbm, s1_hbm, s2_hbm,
          idx_v, *rest):
        rows = rest[:_NB]
        acc = rest[_NB]
        lsem = rest[_NB + 1:_NB + 1 + _NB]
        asem = rest[_NB + 1 + _NB:]
        c = lax.axis_index("c")
        s = lax.axis_index("s")
        wid = s * _NC + c
        base = wid * epw
        mine = pl.ds(s * rpt, rpt)
        tail = pl.ds(_NS * rpt, rem)
        pltpu.sync_copy(dst_hbm.at[wid], idx_v)

        def phase(u_hbm, out_hbm):
            pltpu.sync_copy(z_hbm.at[mine], acc.at[mine])
            @pl.when(s == _NS - 1)
            def _zero_tail():
                pltpu.sync_copy(z_hbm.at[tail], acc.at[tail])
            plsc.subcore_barrier()

            def burst(i0, nb):
                loads = [pltpu.async_copy(
                    u_hbm.at[pl.ds(base + (i0 + b) * _CS, _CS)],
                    rows[b], lsem[b]) for b in range(nb)]
                adds = []
                for b in range(nb):
                    loads[b].wait()
                    adds.append(pltpu.async_copy(
                        rows[b], acc.at[idx_v.at[i0 + b]], asem[b],
                        add=True))
                for d in adds:
                    d.wait()

            def grp(p, _):
                burst(p * _NB, _NB)
                return _

            lax.fori_loop(0, ngrp, grp, 0)
            if nleft:
                burst(ngrp * _NB, nleft)
            plsc.subcore_barrier()
            pltpu.sync_copy(acc.at[mine], out_hbm.at[c].at[mine])
            @pl.when(s == _NS - 1)
            def _write_tail():
                pltpu.sync_copy(acc.at[tail], out_hbm.at[c].at[tail])
            plsc.subcore_barrier()

        phase(u1_hbm, s1_hbm)
        phase(u2_hbm, s2_hbm)

    return k(u1, u2, dst3, zeros_nw)


def _silu(v):
    return v * jax.nn.sigmoid(v)


def _edge_tc(ts, td, dist3, We1r, be1r, We2, be2r, Wn1s, Wn1d, Wn1e, bn1r,
             Wc1s, Wc1d, Wc1e, bc1r, Wc2r, e, be):
    g = e // be

    def unpack(pk):
        pu = jax.lax.bitcast_convert_type(pk, jnp.uint32)
        hi = jax.lax.bitcast_convert_type(
            (pu >> 16).astype(jnp.uint16), jnp.bfloat16)
        lo = jax.lax.bitcast_convert_type(
            pu.astype(jnp.uint16), jnp.bfloat16)
        return jnp.concatenate([hi, lo], axis=1)

    def body(ts_ref, td_ref, d_ref, we1_ref, be1_ref, we2_ref, be2_ref,
             wn1s_ref, wn1d_ref, wn1e_ref, bn1_ref,
             wc1s_ref, wc1d_ref, wc1e_ref, bc1_ref, wc2_ref,
             u1_ref, u2_ref):
        d = d_ref[0, 0, :]
        e1 = d[:, None] * we1_ref[0, :][None, :] + be1_ref[0, :]
        ea = jnp.dot(_silu(e1), we2_ref[...],
                     preferred_element_type=jnp.float32) + be2_ref[0, :]
        hs = unpack(ts_ref[:, 0:64])
        hd = unpack(td_ref[:, 0:64])
        pre_n = (jnp.dot(hs, wn1s_ref[...], preferred_element_type=jnp.float32)
                 + jnp.dot(hd, wn1d_ref[...], preferred_element_type=jnp.float32)
                 + jnp.dot(ea, wn1e_ref[...], preferred_element_type=jnp.float32)
                 + bn1_ref[0, :])
        u1_ref[...] = _silu(pre_n)
        pre_c = (jnp.dot(hs, wc1s_ref[...], preferred_element_type=jnp.float32)
                 + jnp.dot(hd, wc1d_ref[...], preferred_element_type=jnp.float32)
                 + jnp.dot(ea, wc1e_ref[...], preferred_element_type=jnp.float32)
                 + bc1_ref[0, :])
        u = _silu(pre_c)
        cw = jnp.sum(u * wc2_ref[0, :][None, :], axis=1, keepdims=True)
        dvec = ts_ref[:, 64:67] - td_ref[:, 64:67]
        dlen = jnp.maximum(
            jnp.sqrt(jnp.sum(dvec * dvec, axis=1, keepdims=True)), 1e-8)
        cu = cw * (dvec / dlen)
        ones = jnp.ones((be, 1), jnp.float32)
        pad = jnp.zeros((be, _WU - 4), jnp.float32)
        u2_ref[...] = jnp.concatenate([cu, ones, pad], axis=1)

    full = lambda shape: pl.BlockSpec(shape, lambda i: (0,) * len(shape))
    return pl.pallas_call(
        body,
        grid=(g,),
        in_specs=[
            pl.BlockSpec((be, _WT), lambda i: (i, 0)),
            pl.BlockSpec((be, _WT), lambda i: (i, 0)),
            pl.BlockSpec((1, 1, be), lambda i: (i, 0, 0)),
            full((1, 16)), full((1, 16)), full((16, 16)), full((1, 16)),
            full((128, 128)), full((128, 128)), full((16, 128)), full((1, 128)),
            full((128, 128)), full((128, 128)), full((16, 128)), full((1, 128)),
            full((1, 128)),
        ],
        out_specs=[
            pl.BlockSpec((be, _WU), lambda i: (i, 0)),
            pl.BlockSpec((be, _WU), lambda i: (i, 0)),
        ],
        out_shape=[
            jax.ShapeDtypeStruct((e, _WU), jnp.float32),
            jax.ShapeDtypeStruct((e, _WU), jnp.float32),
        ],
    )(ts, td, dist3, We1r, be1r, We2, be2r, Wn1s, Wn1d, Wn1e, bn1r,
      Wc1s, Wc1d, Wc1e, bc1r, Wc2r)


def _final_tc(h, x3, s1s, s2s, Wn2, bn2r, n, bn):
    g = n // bn
    np_ = len(s1s)

    def body(*refs):
        h_ref, x_ref = refs[0], refs[1]
        s1_refs = refs[2:2 + np_]
        s2_refs = refs[2 + np_:2 + 2 * np_]
        wn2_ref, bn2_ref, oh_ref, ox_ref = refs[2 + 2 * np_:]
        hidden = s1_refs[0][0] + s1_refs[0][1]
        s2 = s2_refs[0][0] + s2_refs[0][1]
        for r in s1_refs[1:]:
            hidden = hidden + r[0] + r[1]
        for r in s2_refs[1:]:
            s2 = s2 + r[0] + r[1]
        deg = s2[:, 3:4]
        xa = s2[:, 0:3]
        oh_ref[...] = (h_ref[...]
                       + jnp.dot(hidden, wn2_ref[...],
                                 preferred_element_type=jnp.float32)
                       + deg * bn2_ref[0, :])
        ox_ref[...] = x_ref[...] + xa

    part_spec = pl.BlockSpec((_NC, bn, _WU), lambda i: (0, i, 0))
    return pl.pallas_call(
        body,
        grid=(g,),
        in_specs=[
            pl.BlockSpec((bn, 128), lambda i: (i, 0)),
            pl.BlockSpec((bn, 3), lambda i: (i, 0)),
        ] + [part_spec] * (2 * np_) + [
            pl.BlockSpec((128, 128), lambda i: (0, 0)),
            pl.BlockSpec((1, 128), lambda i: (0, 0)),
        ],
        out_specs=[
            pl.BlockSpec((bn, 128), lambda i: (i, 0)),
            pl.BlockSpec((bn, 3), lambda i: (i, 0)),
        ],
        out_shape=[
            jax.ShapeDtypeStruct((n, 128), jnp.float32),
            jax.ShapeDtypeStruct((n, 3), jnp.float32),
        ],
    )(h, x3, *s1s, *s2s, Wn2, bn2r)


@jax.jit
def kernel(h, x, edge_index, edge_dist, We1, be1, We2, be2, Wn1, bn1, Wn2,
           bn2, Wc1, bc1, Wc2):
    n, nd = h.shape
    e = edge_dist.shape[0]
    e2 = e // _K
    epw = e2 // _NW
    assert nd == 128 and e2 % _NW == 0 and epw % 8 == 0
    assert epw % _CS == 0 and n % _NS == 0

    src = edge_index[0].reshape(_K, e2)
    dst = edge_index[1].reshape(_K, e2)
    src3g = src.reshape(_K, _NW, epw)
    dst3g = dst.reshape(_K, _NW, epw)
    dst3s = dst.reshape(_K, _NW, epw // _CS, _CS)

    hb = h.astype(jnp.bfloat16).reshape(n, 64, 2)
    hi16 = jax.lax.bitcast_convert_type(hb[:, :, 0], jnp.uint16)
    lo16 = jax.lax.bitcast_convert_type(hb[:, :, 1], jnp.uint16)
    packed = jax.lax.bitcast_convert_type(
        (hi16.astype(jnp.uint32) << 16) | lo16.astype(jnp.uint32),
        jnp.float32)
    t = jnp.concatenate(
        [packed, x, jnp.zeros((n, _WT - 67), jnp.float32)], axis=1)

    perm = jnp.concatenate(
        [jnp.arange(0, 128, 2), jnp.arange(1, 128, 2)])
    bf = jnp.bfloat16
    be = 3200
    dist3 = edge_dist.reshape(_K, e2 // be, 1, be)
    zeros_nw = jnp.zeros((n, _WU), jnp.float32)

    def chain(idx, tok):
        return idx + (tok * 0.0).astype(jnp.int32)

    gathered = []
    tok = None
    for k in range(_K):
        s3 = src3g[k] if tok is None else chain(src3g[k], tok)
        ts, td = _gather_sc(t, s3, dst3g[k], n, e2)
        tok = ts[0, 0]
        gathered.append((ts, td))
    parts = []
    for k in range(_K):
        ts, td = gathered[k]
        u1, u2 = _edge_tc(
            ts, td, dist3[k],
            We1.reshape(1, 16), be1.reshape(1, 16), We2, be2.reshape(1, 16),
            Wn1[0:128][perm].astype(bf), Wn1[128:256][perm].astype(bf),
            Wn1[256:272], bn1.reshape(1, 128),
            Wc1[0:128][perm].astype(bf), Wc1[128:256][perm].astype(bf),
            Wc1[256:272], bc1.reshape(1, 128),
            Wc2.reshape(1, 128), e2, be)
        p = _scatter_sc(u1, u2, chain(dst3s[k], tok), zeros_nw, n, e2)
        tok = p[0][0, 0, 0]
        parts.append(p)

    oh, ox = _final_tc(h, x, [p[0] for p in parts], [p[1] for p in parts],
                       Wn2, bn2.reshape(1, 128), n, 1000)
    return oh, ox


# speedup vs baseline: 6.6711x; 1.0104x over previous
import functools

import jax
import jax.numpy as jnp
from jax import lax
from jax.experimental import pallas as pl
from jax.experimental.pallas import tpu as pltpu
from jax.experimental.pallas import tpu_sc as plsc

_NC = 2
_NS = 16
_NW = _NC * _NS

_K = 2
_WT = 128
_WU = 128
_CH = 80
_CS = 40
_NB = 4
_NS_B = 5


def _sc_mesh():
    return plsc.VectorSubcoreMesh(
        core_axis_name="c", subcore_axis_name="s",
        num_cores=_NC, num_subcores=_NS)


def _gather_sc(t, src2, dst2, n, e):
    epw = e // _NW
    nfull = epw // _CH
    tl = epw - nfull * _CH

    @functools.partial(
        pl.kernel,
        out_type=(
            jax.ShapeDtypeStruct((e, _WT), jnp.float32),
            jax.ShapeDtypeStruct((e, _WT), jnp.float32),
        ),
        mesh=_sc_mesh(),
        scratch_types=[
            pltpu.VMEM((epw,), jnp.int32),
            pltpu.VMEM((epw,), jnp.int32),
        ] + [pltpu.VMEM((_CH, _WT), jnp.float32)] * _NB
          + [pltpu.SemaphoreType.DMA] * (2 * _NB),
    )
    def k(t_hbm, src_hbm, dst_hbm, ts_hbm, td_hbm, idxs_v, idxd_v, *rest):
        rows = rest[:_NB]
        gsem = rest[_NB:2 * _NB]
        ssem = rest[2 * _NB:]
        wid = lax.axis_index("s") * _NC + lax.axis_index("c")
        base = wid * epw
        pltpu.sync_copy(src_hbm.at[wid], idxs_v)
        pltpu.sync_copy(dst_hbm.at[wid], idxd_v)

        def burst(i0, nch, sz):
            jobs = [(side, i0 + j)
                    for j in range(nch) for side in (0, 1)]
            gets = []
            for b, (side, i) in enumerate(jobs):
                ih = idxs_v if side == 0 else idxd_v
                iv = pl.ds(i * _CH, sz)
                gets.append(pltpu.async_copy(
                    t_hbm.at[ih.at[iv]], rows[b].at[pl.ds(0, sz)], gsem[b]))
            puts = []
            for b, (side, i) in enumerate(jobs):
                oh = ts_hbm if side == 0 else td_hbm
                gets[b].wait()
                puts.append(pltpu.async_copy(
                    rows[b].at[pl.ds(0, sz)],
                    oh.at[pl.ds(base + i * _CH, sz)], ssem[b]))
            for d in puts:
                d.wait()

        npc = _NB // 2
        def grp(p, _):
            burst(p * npc, npc, _CH)
            return _

        lax.fori_loop(0, nfull // npc, grp, 0)
        for i in range(nfull - nfull % npc, nfull):
            burst(i, 1, _CH)
        if tl:
            burst(nfull, 1, tl)

    return k(t, src2, dst2)


def _scatter_sc(u1, u2, dst3, zeros_nw, n, e):
    epw = e // _NW
    cpt = epw // _CS
    ngrp = cpt // _NS_B
    nleft = cpt - ngrp * _NS_B
    rpt = (n // _NS) // 8 * 8
    rem = n - _NS * rpt

    @functools.partial(
        pl.kernel,
        out_type=(
            jax.ShapeDtypeStruct((_NC, n, _WU), jnp.float32),
            jax.ShapeDtypeStruct((_NC, n, _WU), jnp.float32),
        ),
        mesh=_sc_mesh(),
        scratch_types=[
            pltpu.VMEM((cpt, _CS), jnp.int32),
        ] + [pltpu.VMEM((_CS, _WU), jnp.float32)] * _NS_B + [
            pltpu.VMEM_SHARED((n, _WU), jnp.float32),
        ] + [pltpu.SemaphoreType.DMA] * (2 * _NS_B),
    )
    def k(u1_hbm, u2_hbm, dst_hbm, z_hbm, s1_hbm, s2_hbm,
          idx_v, *rest):
        rows = rest[:_NS_B]
        acc = rest[_NS_B]
        lsem = rest[_NS_B + 1:_NS_B + 1 + _NS_B]
        asem = rest[_NS_B + 1 + _NS_B:]
        c = lax.axis_index("c")
        s = lax.axis_index("s")
        wid = s * _NC + c
        base = wid * epw
        mine = pl.ds(s * rpt, rpt)
        tail = pl.ds(_NS * rpt, rem)
        pltpu.sync_copy(dst_hbm.at[wid], idx_v)

        def phase(u_hbm, out_hbm):
            pltpu.sync_copy(z_hbm.at[mine], acc.at[mine])
            @pl.when(s == _NS - 1)
            def _zero_tail():
                pltpu.sync_copy(z_hbm.at[tail], acc.at[tail])
            plsc.subcore_barrier()

            def burst(i0, nb):
                loads = [pltpu.async_copy(
                    u_hbm.at[pl.ds(base + (i0 + b) * _CS, _CS)],
                    rows[b], lsem[b]) for b in range(nb)]
                adds = []
                for b in range(nb):
                    loads[b].wait()
                    adds.append(pltpu.async_copy(
                        rows[b], acc.at[idx_v.at[i0 + b]], asem[b],
                        add=True))
                for d in adds:
                    d.wait()

            def grp(p, _):
                burst(p * _NS_B, _NS_B)
                return _

            lax.fori_loop(0, ngrp, grp, 0)
            if nleft:
                burst(ngrp * _NS_B, nleft)
            plsc.subcore_barrier()
            pltpu.sync_copy(acc.at[mine], out_hbm.at[c].at[mine])
            @pl.when(s == _NS - 1)
            def _write_tail():
                pltpu.sync_copy(acc.at[tail], out_hbm.at[c].at[tail])
            plsc.subcore_barrier()

        phase(u1_hbm, s1_hbm)
        phase(u2_hbm, s2_hbm)

    return k(u1, u2, dst3, zeros_nw)


def _silu(v):
    return v * jax.nn.sigmoid(v)


def _edge_tc(ts, td, dist3, We1r, be1r, We2, be2r, Wn1s, Wn1d, Wn1e, bn1r,
             Wc1s, Wc1d, Wc1e, bc1r, Wc2r, e, be):
    g = e // be

    def unpack(pk):
        pu = jax.lax.bitcast_convert_type(pk, jnp.uint32)
        hi = jax.lax.bitcast_convert_type(
            (pu >> 16).astype(jnp.uint16), jnp.bfloat16)
        lo = jax.lax.bitcast_convert_type(
            pu.astype(jnp.uint16), jnp.bfloat16)
        return jnp.concatenate([hi, lo], axis=1)

    def body(ts_ref, td_ref, d_ref, we1_ref, be1_ref, we2_ref, be2_ref,
             wn1s_ref, wn1d_ref, wn1e_ref, bn1_ref,
             wc1s_ref, wc1d_ref, wc1e_ref, bc1_ref, wc2_ref,
             u1_ref, u2_ref):
        d = d_ref[0, 0, :]
        e1 = d[:, None] * we1_ref[0, :][None, :] + be1_ref[0, :]
        ea = jnp.dot(_silu(e1), we2_ref[...],
                     preferred_element_type=jnp.float32) + be2_ref[0, :]
        hs = unpack(ts_ref[:, 0:64])
        hd = unpack(td_ref[:, 0:64])
        pre_n = (jnp.dot(hs, wn1s_ref[...], preferred_element_type=jnp.float32)
                 + jnp.dot(hd, wn1d_ref[...], preferred_element_type=jnp.float32)
                 + jnp.dot(ea, wn1e_ref[...], preferred_element_type=jnp.float32)
                 + bn1_ref[0, :])
        u1_ref[...] = _silu(pre_n)
        pre_c = (jnp.dot(hs, wc1s_ref[...], preferred_element_type=jnp.float32)
                 + jnp.dot(hd, wc1d_ref[...], preferred_element_type=jnp.float32)
                 + jnp.dot(ea, wc1e_ref[...], preferred_element_type=jnp.float32)
                 + bc1_ref[0, :])
        u = _silu(pre_c)
        cw = jnp.sum(u * wc2_ref[0, :][None, :], axis=1, keepdims=True)
        dvec = ts_ref[:, 64:67] - td_ref[:, 64:67]
        dlen = jnp.maximum(
            jnp.sqrt(jnp.sum(dvec * dvec, axis=1, keepdims=True)), 1e-8)
        cu = cw * (dvec / dlen)
        ones = jnp.ones((be, 1), jnp.float32)
        pad = jnp.zeros((be, _WU - 4), jnp.float32)
        u2_ref[...] = jnp.concatenate([cu, ones, pad], axis=1)

    full = lambda shape: pl.BlockSpec(shape, lambda i: (0,) * len(shape))
    return pl.pallas_call(
        body,
        grid=(g,),
        in_specs=[
            pl.BlockSpec((be, _WT), lambda i: (i, 0)),
            pl.BlockSpec((be, _WT), lambda i: (i, 0)),
            pl.BlockSpec((1, 1, be), lambda i: (i, 0, 0)),
            full((1, 16)), full((1, 16)), full((16, 16)), full((1, 16)),
            full((128, 128)), full((128, 128)), full((16, 128)), full((1, 128)),
            full((128, 128)), full((128, 128)), full((16, 128)), full((1, 128)),
            full((1, 128)),
        ],
        out_specs=[
            pl.BlockSpec((be, _WU), lambda i: (i, 0)),
            pl.BlockSpec((be, _WU), lambda i: (i, 0)),
        ],
        out_shape=[
            jax.ShapeDtypeStruct((e, _WU), jnp.float32),
            jax.ShapeDtypeStruct((e, _WU), jnp.float32),
        ],
    )(ts, td, dist3, We1r, be1r, We2, be2r, Wn1s, Wn1d, Wn1e, bn1r,
      Wc1s, Wc1d, Wc1e, bc1r, Wc2r)


def _final_tc(h, x3, s1s, s2s, Wn2, bn2r, n, bn):
    g = n // bn
    np_ = len(s1s)

    def body(*refs):
        h_ref, x_ref = refs[0], refs[1]
        s1_refs = refs[2:2 + np_]
        s2_refs = refs[2 + np_:2 + 2 * np_]
        wn2_ref, bn2_ref, oh_ref, ox_ref = refs[2 + 2 * np_:]
        hidden = s1_refs[0][0] + s1_refs[0][1]
        s2 = s2_refs[0][0] + s2_refs[0][1]
        for r in s1_refs[1:]:
            hidden = hidden + r[0] + r[1]
        for r in s2_refs[1:]:
            s2 = s2 + r[0] + r[1]
        deg = s2[:, 3:4]
        xa = s2[:, 0:3]
        oh_ref[...] = (h_ref[...]
                       + jnp.dot(hidden, wn2_ref[...],
                                 preferred_element_type=jnp.float32)
                       + deg * bn2_ref[0, :])
        ox_ref[...] = x_ref[...] + xa

    part_spec = pl.BlockSpec((_NC, bn, _WU), lambda i: (0, i, 0))
    return pl.pallas_call(
        body,
        grid=(g,),
        in_specs=[
            pl.BlockSpec((bn, 128), lambda i: (i, 0)),
            pl.BlockSpec((bn, 3), lambda i: (i, 0)),
        ] + [part_spec] * (2 * np_) + [
            pl.BlockSpec((128, 128), lambda i: (0, 0)),
            pl.BlockSpec((1, 128), lambda i: (0, 0)),
        ],
        out_specs=[
            pl.BlockSpec((bn, 128), lambda i: (i, 0)),
            pl.BlockSpec((bn, 3), lambda i: (i, 0)),
        ],
        out_shape=[
            jax.ShapeDtypeStruct((n, 128), jnp.float32),
            jax.ShapeDtypeStruct((n, 3), jnp.float32),
        ],
    )(h, x3, *s1s, *s2s, Wn2, bn2r)


@jax.jit
def kernel(h, x, edge_index, edge_dist, We1, be1, We2, be2, Wn1, bn1, Wn2,
           bn2, Wc1, bc1, Wc2):
    n, nd = h.shape
    e = edge_dist.shape[0]
    e2 = e // _K
    epw = e2 // _NW
    assert nd == 128 and e2 % _NW == 0 and epw % 8 == 0
    assert epw % _CS == 0 and n % _NS == 0

    src = edge_index[0].reshape(_K, e2)
    dst = edge_index[1].reshape(_K, e2)
    src3g = src.reshape(_K, _NW, epw)
    dst3g = dst.reshape(_K, _NW, epw)
    dst3s = dst.reshape(_K, _NW, epw // _CS, _CS)

    hb = h.astype(jnp.bfloat16).reshape(n, 64, 2)
    hi16 = jax.lax.bitcast_convert_type(hb[:, :, 0], jnp.uint16)
    lo16 = jax.lax.bitcast_convert_type(hb[:, :, 1], jnp.uint16)
    packed = jax.lax.bitcast_convert_type(
        (hi16.astype(jnp.uint32) << 16) | lo16.astype(jnp.uint32),
        jnp.float32)
    t = jnp.concatenate(
        [packed, x, jnp.zeros((n, _WT - 67), jnp.float32)], axis=1)

    perm = jnp.concatenate(
        [jnp.arange(0, 128, 2), jnp.arange(1, 128, 2)])
    bf = jnp.bfloat16
    be = 3200
    dist3 = edge_dist.reshape(_K, e2 // be, 1, be)
    zeros_nw = jnp.zeros((n, _WU), jnp.float32)

    def chain(idx, tok):
        return idx + (tok * 0.0).astype(jnp.int32)

    gathered = []
    tok = None
    for k in range(_K):
        s3 = src3g[k] if tok is None else chain(src3g[k], tok)
        ts, td = _gather_sc(t, s3, dst3g[k], n, e2)
        tok = ts[0, 0]
        gathered.append((ts, td))
    parts = []
    for k in range(_K):
        ts, td = gathered[k]
        u1, u2 = _edge_tc(
            ts, td, dist3[k],
            We1.reshape(1, 16), be1.reshape(1, 16), We2, be2.reshape(1, 16),
            Wn1[0:128][perm].astype(bf), Wn1[128:256][perm].astype(bf),
            Wn1[256:272], bn1.reshape(1, 128),
            Wc1[0:128][perm].astype(bf), Wc1[128:256][perm].astype(bf),
            Wc1[256:272], bc1.reshape(1, 128),
            Wc2.reshape(1, 128), e2, be)
        p = _scatter_sc(u1, u2, chain(dst3s[k], tok), zeros_nw, n, e2)
        tok = p[0][0, 0, 0]
        parts.append(p)

    oh, ox = _final_tc(h, x, [p[0] for p in parts], [p[1] for p in parts],
                       Wn2, bn2.reshape(1, 128), n, 1000)
    return oh, ox


# speedup vs baseline: 6.7216x; 1.0076x over previous
import functools

import jax
import jax.numpy as jnp
from jax import lax
from jax.experimental import pallas as pl
from jax.experimental.pallas import tpu as pltpu
from jax.experimental.pallas import tpu_sc as plsc

_NC = 2
_NS = 16
_NW = _NC * _NS

_K = 2
_WT = 128
_WU = 128
_CH = 80
_CS = 40
_NB = 6
_NS_B = 5


def _sc_mesh():
    return plsc.VectorSubcoreMesh(
        core_axis_name="c", subcore_axis_name="s",
        num_cores=_NC, num_subcores=_NS)


def _gather_sc(t, src2, dst2, n, e):
    epw = e // _NW
    nfull = epw // _CH
    tl = epw - nfull * _CH

    @functools.partial(
        pl.kernel,
        out_type=(
            jax.ShapeDtypeStruct((e, _WT), jnp.float32),
            jax.ShapeDtypeStruct((e, _WT), jnp.float32),
        ),
        mesh=_sc_mesh(),
        scratch_types=[
            pltpu.VMEM((epw,), jnp.int32),
            pltpu.VMEM((epw,), jnp.int32),
        ] + [pltpu.VMEM((_CH, _WT), jnp.float32)] * _NB
          + [pltpu.SemaphoreType.DMA] * (2 * _NB),
    )
    def k(t_hbm, src_hbm, dst_hbm, ts_hbm, td_hbm, idxs_v, idxd_v, *rest):
        rows = rest[:_NB]
        gsem = rest[_NB:2 * _NB]
        ssem = rest[2 * _NB:]
        wid = lax.axis_index("s") * _NC + lax.axis_index("c")
        base = wid * epw
        pltpu.sync_copy(src_hbm.at[wid], idxs_v)
        pltpu.sync_copy(dst_hbm.at[wid], idxd_v)

        def burst(i0, nch, sz):
            jobs = [(side, i0 + j)
                    for j in range(nch) for side in (0, 1)]
            gets = []
            for b, (side, i) in enumerate(jobs):
                ih = idxs_v if side == 0 else idxd_v
                iv = pl.ds(i * _CH, sz)
                gets.append(pltpu.async_copy(
                    t_hbm.at[ih.at[iv]], rows[b].at[pl.ds(0, sz)], gsem[b]))
            puts = []
            for b, (side, i) in enumerate(jobs):
                oh = ts_hbm if side == 0 else td_hbm
                gets[b].wait()
                puts.append(pltpu.async_copy(
                    rows[b].at[pl.ds(0, sz)],
                    oh.at[pl.ds(base + i * _CH, sz)], ssem[b]))
            for d in puts:
                d.wait()

        npc = _NB // 2
        def grp(p, _):
            burst(p * npc, npc, _CH)
            return _

        lax.fori_loop(0, nfull // npc, grp, 0)
        for i in range(nfull - nfull % npc, nfull):
            burst(i, 1, _CH)
        if tl:
            burst(nfull, 1, tl)

    return k(t, src2, dst2)


def _scatter_sc(u1, u2, dst3, zeros_nw, n, e):
    epw = e // _NW
    cpt = epw // _CS
    ngrp = cpt // _NS_B
    nleft = cpt - ngrp * _NS_B
    rpt = (n // _NS) // 8 * 8
    rem = n - _NS * rpt

    @functools.partial(
        pl.kernel,
        out_type=(
            jax.ShapeDtypeStruct((_NC, n, _WU), jnp.float32),
            jax.ShapeDtypeStruct((_NC, n, _WU), jnp.float32),
        ),
        mesh=_sc_mesh(),
        scratch_types=[
            pltpu.VMEM((cpt, _CS), jnp.int32),
        ] + [pltpu.VMEM((_CS, _WU), jnp.float32)] * _NS_B + [
            pltpu.VMEM_SHARED((n, _WU), jnp.float32),
        ] + [pltpu.SemaphoreType.DMA] * (2 * _NS_B),
    )
    def k(u1_hbm, u2_hbm, dst_hbm, z_hbm, s1_hbm, s2_hbm,
          idx_v, *rest):
        rows = rest[:_NS_B]
        acc = rest[_NS_B]
        lsem = rest[_NS_B + 1:_NS_B + 1 + _NS_B]
        asem = rest[_NS_B + 1 + _NS_B:]
        c = lax.axis_index("c")
        s = lax.axis_index("s")
        wid = s * _NC + c
        base = wid * epw
        mine = pl.ds(s * rpt, rpt)
        tail = pl.ds(_NS * rpt, rem)
        pltpu.sync_copy(dst_hbm.at[wid], idx_v)

        def phase(u_hbm, out_hbm):
            pltpu.sync_copy(z_hbm.at[mine], acc.at[mine])
            @pl.when(s == _NS - 1)
            def _zero_tail():
                pltpu.sync_copy(z_hbm.at[tail], acc.at[tail])
            plsc.subcore_barrier()

            def burst(i0, nb):
                loads = [pltpu.async_copy(
                    u_hbm.at[pl.ds(base + (i0 + b) * _CS, _CS)],
                    rows[b], lsem[b]) for b in range(nb)]
                adds = []
                for b in range(nb):
                    loads[b].wait()
                    adds.append(pltpu.async_copy(
                        rows[b], acc.at[idx_v.at[i0 + b]], asem[b],
                        add=True))
                for d in adds:
                    d.wait()

            def grp(p, _):
                burst(p * _NS_B, _NS_B)
                return _

            lax.fori_loop(0, ngrp, grp, 0)
            if nleft:
                burst(ngrp * _NS_B, nleft)
            plsc.subcore_barrier()
            pltpu.sync_copy(acc.at[mine], out_hbm.at[c].at[mine])
            @pl.when(s == _NS - 1)
            def _write_tail():
                pltpu.sync_copy(acc.at[tail], out_hbm.at[c].at[tail])
            plsc.subcore_barrier()

        phase(u1_hbm, s1_hbm)
        phase(u2_hbm, s2_hbm)

    return k(u1, u2, dst3, zeros_nw)


def _silu(v):
    return v * jax.nn.sigmoid(v)


def _edge_tc(ts, td, dist3, We1r, be1r, We2, be2r, Wn1s, Wn1d, Wn1e, bn1r,
             Wc1s, Wc1d, Wc1e, bc1r, Wc2r, e, be):
    g = e // be

    def unpack(pk):
        pu = jax.lax.bitcast_convert_type(pk, jnp.uint32)
        hi = jax.lax.bitcast_convert_type(
            (pu >> 16).astype(jnp.uint16), jnp.bfloat16)
        lo = jax.lax.bitcast_convert_type(
            pu.astype(jnp.uint16), jnp.bfloat16)
        return jnp.concatenate([hi, lo], axis=1)

    def body(ts_ref, td_ref, d_ref, we1_ref, be1_ref, we2_ref, be2_ref,
             wn1s_ref, wn1d_ref, wn1e_ref, bn1_ref,
             wc1s_ref, wc1d_ref, wc1e_ref, bc1_ref, wc2_ref,
             u1_ref, u2_ref):
        d = d_ref[0, 0, :]
        e1 = d[:, None] * we1_ref[0, :][None, :] + be1_ref[0, :]
        ea = jnp.dot(_silu(e1), we2_ref[...],
                     preferred_element_type=jnp.float32) + be2_ref[0, :]
        hs = unpack(ts_ref[:, 0:64])
        hd = unpack(td_ref[:, 0:64])
        pre_n = (jnp.dot(hs, wn1s_ref[...], preferred_element_type=jnp.float32)
                 + jnp.dot(hd, wn1d_ref[...], preferred_element_type=jnp.float32)
                 + jnp.dot(ea, wn1e_ref[...], preferred_element_type=jnp.float32)
                 + bn1_ref[0, :])
        u1_ref[...] = _silu(pre_n)
        pre_c = (jnp.dot(hs, wc1s_ref[...], preferred_element_type=jnp.float32)
                 + jnp.dot(hd, wc1d_ref[...], preferred_element_type=jnp.float32)
                 + jnp.dot(ea, wc1e_ref[...], preferred_element_type=jnp.float32)
                 + bc1_ref[0, :])
        u = _silu(pre_c)
        cw = jnp.sum(u * wc2_ref[0, :][None, :], axis=1, keepdims=True)
        dvec = ts_ref[:, 64:67] - td_ref[:, 64:67]
        dlen = jnp.maximum(
            jnp.sqrt(jnp.sum(dvec * dvec, axis=1, keepdims=True)), 1e-8)
        cu = cw * (dvec / dlen)
        ones = jnp.ones((be, 1), jnp.float32)
        pad = jnp.zeros((be, _WU - 4), jnp.float32)
        u2_ref[...] = jnp.concatenate([cu, ones, pad], axis=1)

    full = lambda shape: pl.BlockSpec(shape, lambda i: (0,) * len(shape))
    return pl.pallas_call(
        body,
        grid=(g,),
        in_specs=[
            pl.BlockSpec((be, _WT), lambda i: (i, 0)),
            pl.BlockSpec((be, _WT), lambda i: (i, 0)),
            pl.BlockSpec((1, 1, be), lambda i: (i, 0, 0)),
            full((1, 16)), full((1, 16)), full((16, 16)), full((1, 16)),
            full((128, 128)), full((128, 128)), full((16, 128)), full((1, 128)),
            full((128, 128)), full((128, 128)), full((16, 128)), full((1, 128)),
            full((1, 128)),
        ],
        out_specs=[
            pl.BlockSpec((be, _WU), lambda i: (i, 0)),
            pl.BlockSpec((be, _WU), lambda i: (i, 0)),
        ],
        out_shape=[
            jax.ShapeDtypeStruct((e, _WU), jnp.float32),
            jax.ShapeDtypeStruct((e, _WU), jnp.float32),
        ],
    )(ts, td, dist3, We1r, be1r, We2, be2r, Wn1s, Wn1d, Wn1e, bn1r,
      Wc1s, Wc1d, Wc1e, bc1r, Wc2r)


def _final_tc(h, x3, s1s, s2s, Wn2, bn2r, n, bn):
    g = n // bn
    np_ = len(s1s)

    def body(*refs):
        h_ref, x_ref = refs[0], refs[1]
        s1_refs = refs[2:2 + np_]
        s2_refs = refs[2 + np_:2 + 2 * np_]
        wn2_ref, bn2_ref, oh_ref, ox_ref = refs[2 + 2 * np_:]
        hidden = s1_refs[0][0] + s1_refs[0][1]
        s2 = s2_refs[0][0] + s2_refs[0][1]
        for r in s1_refs[1:]:
            hidden = hidden + r[0] + r[1]
        for r in s2_refs[1:]:
            s2 = s2 + r[0] + r[1]
        deg = s2[:, 3:4]
        xa = s2[:, 0:3]
        oh_ref[...] = (h_ref[...]
                       + jnp.dot(hidden, wn2_ref[...],
                                 preferred_element_type=jnp.float32)
                       + deg * bn2_ref[0, :])
        ox_ref[...] = x_ref[...] + xa

    part_spec = pl.BlockSpec((_NC, bn, _WU), lambda i: (0, i, 0))
    return pl.pallas_call(
        body,
        grid=(g,),
        in_specs=[
            pl.BlockSpec((bn, 128), lambda i: (i, 0)),
            pl.BlockSpec((bn, 3), lambda i: (i, 0)),
        ] + [part_spec] * (2 * np_) + [
            pl.BlockSpec((128, 128), lambda i: (0, 0)),
            pl.BlockSpec((1, 128), lambda i: (0, 0)),
        ],
        out_specs=[
            pl.BlockSpec((bn, 128), lambda i: (i, 0)),
            pl.BlockSpec((bn, 3), lambda i: (i, 0)),
        ],
        out_shape=[
            jax.ShapeDtypeStruct((n, 128), jnp.float32),
            jax.ShapeDtypeStruct((n, 3), jnp.float32),
        ],
    )(h, x3, *s1s, *s2s, Wn2, bn2r)


@jax.jit
def kernel(h, x, edge_index, edge_dist, We1, be1, We2, be2, Wn1, bn1, Wn2,
           bn2, Wc1, bc1, Wc2):
    n, nd = h.shape
    e = edge_dist.shape[0]
    e2 = e // _K
    epw = e2 // _NW
    assert nd == 128 and e2 % _NW == 0 and epw % 8 == 0
    assert epw % _CS == 0 and n % _NS == 0

    src = edge_index[0].reshape(_K, e2)
    dst = edge_index[1].reshape(_K, e2)
    src3g = src.reshape(_K, _NW, epw)
    dst3g = dst.reshape(_K, _NW, epw)
    dst3s = dst.reshape(_K, _NW, epw // _CS, _CS)

    hb = h.astype(jnp.bfloat16).reshape(n, 64, 2)
    hi16 = jax.lax.bitcast_convert_type(hb[:, :, 0], jnp.uint16)
    lo16 = jax.lax.bitcast_convert_type(hb[:, :, 1], jnp.uint16)
    packed = jax.lax.bitcast_convert_type(
        (hi16.astype(jnp.uint32) << 16) | lo16.astype(jnp.uint32),
        jnp.float32)
    t = jnp.concatenate(
        [packed, x, jnp.zeros((n, _WT - 67), jnp.float32)], axis=1)

    perm = jnp.concatenate(
        [jnp.arange(0, 128, 2), jnp.arange(1, 128, 2)])
    bf = jnp.bfloat16
    be = 3200
    dist3 = edge_dist.reshape(_K, e2 // be, 1, be)
    zeros_nw = jnp.zeros((n, _WU), jnp.float32)

    def chain(idx, tok):
        return idx + (tok * 0.0).astype(jnp.int32)

    gathered = []
    tok = None
    for k in range(_K):
        s3 = src3g[k] if tok is None else chain(src3g[k], tok)
        ts, td = _gather_sc(t, s3, dst3g[k], n, e2)
        tok = ts[0, 0]
        gathered.append((ts, td))
    parts = []
    for k in range(_K):
        ts, td = gathered[k]
        u1, u2 = _edge_tc(
            ts, td, dist3[k],
            We1.reshape(1, 16), be1.reshape(1, 16), We2, be2.reshape(1, 16),
            Wn1[0:128][perm].astype(bf), Wn1[128:256][perm].astype(bf),
            Wn1[256:272], bn1.reshape(1, 128),
            Wc1[0:128][perm].astype(bf), Wc1[128:256][perm].astype(bf),
            Wc1[256:272], bc1.reshape(1, 128),
            Wc2.reshape(1, 128), e2, be)
        p = _scatter_sc(u1, u2, chain(dst3s[k], tok), zeros_nw, n, e2)
        tok = p[0][0, 0, 0]
        parts.append(p)

    oh, ox = _final_tc(h, x, [p[0] for p in parts], [p[1] for p in parts],
                       Wn2, bn2.reshape(1, 128), n, 1000)
    return oh, ox
